# baseline (speedup 1.0000x reference)
import jax
import jax.numpy as jnp
from jax import lax
from jax.experimental import pallas as pl
from jax.experimental.pallas import tpu as pltpu
from jax.experimental.pallas import tpu_sc as plsc

_CH = 1024
_PER_W = 57720


def _sc_body(x_hbm, tbl_hbm, out_hbm, xv, tblv, ov, sem):
    NC = 2
    wid = lax.axis_index("s") * NC + lax.axis_index("c")
    tok0 = wid * _PER_W
    iota = lax.iota(jnp.int32, 16)
    iota6 = iota * 6
    iota24 = iota * 24

    pltpu.sync_copy(tbl_hbm, tblv)

    nsteps = (_PER_W + _CH - 1) // _CH

    def step(i, carry):
        off = i * _CH
        off = lax.select(off > _PER_W - _CH, _PER_W - _CH, off)
        start = tok0 + off
        pltpu.sync_copy(x_hbm.at[pl.ds(start * 6, _CH * 6)], xv)

        def group(j, carry2):
            p = iota6 + j * 96
            i0 = plsc.load_gather(xv, [p + 3])
            i1 = plsc.load_gather(xv, [p + 4])
            i2 = plsc.load_gather(xv, [p + 5])
            comb24 = (i0 * 49 + i1 * 7 + i2) * 24
            obase = iota24 + j * 384
            for c in range(24):
                val = plsc.load_gather(tblv, [comb24 + c])
                plsc.store_scatter(ov, [obase + c], val)
            return carry2

        lax.fori_loop(0, _CH // 16, group, 0)
        pltpu.sync_copy(ov, out_hbm.at[pl.ds(start * 24, _CH * 24)])
        return carry

    lax.fori_loop(0, nsteps, step, 0)


def kernel(X, W0, W1, W2):
    B, N, T, F = X.shape
    M = B * N * T
    Xf = X.reshape(M * F)

    c = jnp.arange(343)
    tbl = jnp.concatenate(
        [W0[c // 49], W1[(c // 7) % 7], W2[c % 7]], axis=1).reshape(-1)

    mesh = plsc.VectorSubcoreMesh(core_axis_name="c", subcore_axis_name="s")
    out = pl.kernel(
        _sc_body,
        mesh=mesh,
        compiler_params=pltpu.CompilerParams(needs_layout_passes=False),
        out_type=jax.ShapeDtypeStruct((M * 24,), jnp.float32),
        scratch_types=[
            pltpu.VMEM((_CH * F,), jnp.int32),
            pltpu.VMEM((343 * 24,), jnp.float32),
            pltpu.VMEM((_CH * 24,), jnp.float32),
            pltpu.SemaphoreType.DMA,
        ],
    )(Xf, tbl)
    return out.reshape(B, N, T, 24)


# speedup vs baseline: 40.9495x; 40.9495x over previous
import jax
import jax.numpy as jnp
from jax.experimental import pallas as pl
from jax.experimental.pallas import tpu as pltpu


def _body(x_ref, v_ref, o_ref):
    Bb = o_ref.shape[0]
    Nb = o_ref.shape[3]
    for b in range(Bb):
        for g in range(3):
            idxp = x_ref[b, g]
            cands = [jnp.broadcast_to(v_ref[8 * g:8 * g + 8, k:k + 1], (8, Nb))
                     for k in range(7)]
            for t in range(24):
                idx = jnp.broadcast_to(idxp[t:t + 1, :], (8, Nb))
                acc = cands[0]
                for k in range(1, 7):
                    acc = jnp.where(idx == k, cands[k], acc)
                o_ref[b, t, 8 * g:8 * g + 8, :] = acc


def kernel(X, W0, W1, W2):
    B, N, T, F = X.shape
    Xt = jnp.transpose(X, (0, 3, 2, 1))

    Tt = jnp.concatenate([W0[:7], W1[:7], W2[:7]], axis=1)
    Vt = jnp.concatenate([Tt.T, jnp.zeros((24, 1), jnp.float32)], axis=1)

    Nb = 2432
    Bb = 2
    grid = (B // Bb, pl.cdiv(N, Nb))
    out = pl.pallas_call(
        _body,
        grid=grid,
        in_specs=[
            pl.BlockSpec((Bb, 3, T, Nb), lambda b, i: (b, 1, 0, i)),
            pl.BlockSpec((T, 8), lambda b, i: (0, 0)),
        ],
        out_specs=pl.BlockSpec((Bb, T, 24, Nb), lambda b, i: (b, 0, 0, i)),
        out_shape=jax.ShapeDtypeStruct((B, T, 24, N), jnp.float32),
        compiler_params=pltpu.CompilerParams(
            dimension_semantics=("parallel", "parallel"),
        ),
    )(Xt, Vt)
    return jnp.transpose(out, (0, 3, 1, 2))
